# baseline (speedup 1.0000x reference)
import functools

import jax
import jax.numpy as jnp
from jax import lax
from jax.experimental import pallas as pl
from jax.experimental.pallas import tpu as pltpu
from jax.experimental.pallas import tpu_sc as plsc

_F = 26
_V = 100000
_E = 32
_B = 16384
_NC, _NS = 2, 16
_NW = _NC * _NS
_BF = _B * _F
_BPW = _BF // _NW
_C = 128
_NCH = _BPW // _C

_mesh = plsc.VectorSubcoreMesh(core_axis_name="c", subcore_axis_name="s")


@functools.partial(
    pl.kernel,
    out_type=jax.ShapeDtypeStruct((_BF, _E), jnp.float32),
    mesh=_mesh,
    scratch_types=[
        pltpu.VMEM((_NCH, _C), jnp.int32),
        pltpu.VMEM((_C, _E), jnp.float32),
        pltpu.SemaphoreType.DMA,
    ],
    compiler_params=pltpu.CompilerParams(use_tc_tiling_on_sc=False),
)
def _sc_gather(tab_hbm, idx_hbm, out_hbm, idx_v, rows_v, sem):
    wid = lax.axis_index("s") * _NC + lax.axis_index("c")
    base = wid * _BPW
    pltpu.sync_copy(idx_hbm.at[wid], idx_v)

    def body(j, carry):
        pltpu.async_copy(tab_hbm.at[idx_v.at[j]], rows_v, sem).wait()
        pltpu.sync_copy(rows_v, out_hbm.at[pl.ds(base + j * _C, _C)])
        return carry

    lax.fori_loop(0, _NCH, body, 0)


def kernel(families, tables):
    flat_tab = tables.reshape(_F * _V, _E)
    offs = (jnp.arange(_F, dtype=jnp.int32) * _V)[None, :]
    flat_idx = (families.astype(jnp.int32).T + offs).reshape(_NW, _NCH, _C)
    out = _sc_gather(flat_tab, flat_idx)
    return out.reshape(_B, _F * _E)


# speedup vs baseline: 1.0354x; 1.0354x over previous
import functools

import jax
import jax.numpy as jnp
from jax import lax
from jax.experimental import pallas as pl
from jax.experimental.pallas import tpu as pltpu
from jax.experimental.pallas import tpu_sc as plsc

_F = 26
_V = 100000
_E = 32
_B = 16384
_NC, _NS = 2, 16
_NW = _NC * _NS
_BF = _B * _F
_BPW = _BF // _NW
_C = 128
_NCH = _BPW // _C
_NBUF = 2

_mesh = plsc.VectorSubcoreMesh(core_axis_name="c", subcore_axis_name="s")


@functools.partial(
    pl.kernel,
    out_type=jax.ShapeDtypeStruct((_BF, _E), jnp.float32),
    mesh=_mesh,
    scratch_types=[
        pltpu.VMEM((_NCH, _C), jnp.int32),
        pltpu.VMEM((_NBUF, _C, _E), jnp.float32),
        pltpu.SemaphoreType.DMA((_NBUF,)),
    ],
    compiler_params=pltpu.CompilerParams(use_tc_tiling_on_sc=False),
)
def _sc_gather(tab_hbm, idx_hbm, out_hbm, idx_v, rows_v, sem):
    wid = lax.axis_index("s") * _NC + lax.axis_index("c")
    base = wid * _BPW
    pltpu.sync_copy(idx_hbm.at[wid], idx_v)

    def gather_desc(j, slot):
        return pltpu.make_async_copy(
            tab_hbm.at[idx_v.at[j]], rows_v.at[slot], sem.at[slot])

    for s in range(_NBUF - 1):
        gather_desc(s, s).start()

    def body(j, carry):
        slot = lax.rem(j, _NBUF)
        nxt = j + (_NBUF - 1)

        @pl.when(nxt < _NCH)
        def _():
            gather_desc(nxt, lax.rem(nxt, _NBUF)).start()

        gather_desc(j, slot).wait()
        pltpu.sync_copy(rows_v.at[slot], out_hbm.at[pl.ds(base + j * _C, _C)])
        return carry

    lax.fori_loop(0, _NCH, body, 0)


def kernel(families, tables):
    flat_tab = tables.reshape(_F * _V, _E)
    offs = (jnp.arange(_F, dtype=jnp.int32) * _V)[None, :]
    flat_idx = (families.astype(jnp.int32).T + offs).reshape(_NW, _NCH, _C)
    out = _sc_gather(flat_tab, flat_idx)
    return out.reshape(_B, _F * _E)


# speedup vs baseline: 3.3778x; 3.2622x over previous
import functools

import jax
import jax.numpy as jnp
from jax import lax
from jax.experimental import pallas as pl
from jax.experimental.pallas import tpu as pltpu
from jax.experimental.pallas import tpu_sc as plsc

_F = 26
_V = 100000
_E = 32
_B = 16384
_NC, _NS = 2, 16
_NW = _NC * _NS
_Q = _B // 4

_mesh = plsc.VectorSubcoreMesh(core_axis_name="c", subcore_axis_name="s")


@functools.partial(
    pl.kernel,
    out_type=jax.ShapeDtypeStruct((_F * _E, _B), jnp.float32),
    mesh=_mesh,
    scratch_types=[
        pltpu.VMEM((_V,), jnp.float32),
        pltpu.VMEM((_B,), jnp.int32),
        pltpu.VMEM((2, _Q), jnp.float32),
        pltpu.SemaphoreType.DMA,
        pltpu.SemaphoreType.DMA((2,)),
    ],
    compiler_params=pltpu.CompilerParams(needs_layout_passes=False),
)
def _sc_lookup(tab_hbm, fam_hbm, out_hbm, row_v, idx_v, col_v, in_sem, out_sem):
    w = lax.axis_index("s") * _NC + lax.axis_index("c")

    def per_field(f, carry):
        c = f * _E + w
        row_cp = pltpu.make_async_copy(tab_hbm.at[f, w], row_v, in_sem)
        idx_cp = pltpu.make_async_copy(fam_hbm.at[f], idx_v, in_sem)
        row_cp.start()
        idx_cp.start()
        row_cp.wait()
        idx_cp.wait()

        def out_desc(q, slot):
            return pltpu.make_async_copy(
                col_v.at[slot], out_hbm.at[c, pl.ds(q * _Q, _Q)], out_sem.at[slot])

        def per_quarter(q, carry2):
            slot = lax.rem(q, 2)

            @pl.when(q >= 2)
            def _():
                out_desc(q - 2, slot).wait()

            @pl.loop(0, _Q // 16)
            def _gather(i):
                vidx = idx_v[pl.ds(q * _Q + i * 16, 16)]
                col_v[slot, pl.ds(i * 16, 16)] = plsc.load_gather(row_v, [vidx])

            out_desc(q, slot).start()
            return carry2

        lax.fori_loop(0, 4, per_quarter, 0)
        out_desc(2, 0).wait()
        out_desc(3, 1).wait()
        return carry

    lax.fori_loop(0, _F, per_field, 0)


def kernel(families, tables):
    tab_t = jnp.transpose(tables, (0, 2, 1))
    out_t = _sc_lookup(tab_t, families.astype(jnp.int32))
    return out_t.T
